# baseline (device time: 11796 ns/iter reference)
import jax
import jax.numpy as jnp
from jax import lax
from jax.experimental import pallas as pl
from jax.experimental.pallas import tpu as pltpu

S = 4


def kernel(x):
    m, n = x.shape
    half = m // 2
    sub = half // S

    def body(x_ref, out_ref, x_send_sems, x_recv_sems, y_send_sems, y_recv_sems):
        mx = lax.axis_index("x")
        my = lax.axis_index("y")
        mz = lax.axis_index("z")
        x_peer = (1 - mx, my, mz)
        y_peer = (mx, 1 - my, mz)

        barrier_sem = pltpu.get_barrier_semaphore()
        for peer in (x_peer, y_peer):
            pl.semaphore_signal(
                barrier_sem, inc=1, device_id=peer,
                device_id_type=pl.DeviceIdType.MESH,
            )
        pl.semaphore_wait(barrier_sem, 2)

        own_base = mx * m
        other_base = (1 - mx) * m
        send_off = my * half
        fwd_off = send_off
        recv_off = half - send_off

        def rows(base, s):
            return (pl.ds(base + s * sub, sub), slice(None))

        x_send_d, x_recv_d, y_send_d, y_recv_d = [], [], [], []
        for s in range(S):
            out_ref[rows(own_base + send_off, s)] = x_ref[
                rows(send_off, s)[0], :
            ].astype(jnp.bfloat16)
            d = pltpu.make_async_remote_copy(
                src_ref=out_ref.at[rows(own_base + send_off, s)],
                dst_ref=out_ref.at[rows(own_base + send_off, s)],
                send_sem=x_send_sems.at[s],
                recv_sem=x_recv_sems.at[s],
                device_id=x_peer,
                device_id_type=pl.DeviceIdType.MESH,
            )
            d.start()
            x_send_d.append(d)

        for s in range(S):
            x_recv_d.append(pltpu.make_async_remote_copy(
                src_ref=out_ref.at[rows(own_base + send_off, s)],
                dst_ref=out_ref.at[rows(other_base + fwd_off, s)],
                send_sem=x_send_sems.at[s],
                recv_sem=x_recv_sems.at[s],
                device_id=x_peer,
                device_id_type=pl.DeviceIdType.MESH,
            ))
            y_recv_d.append(pltpu.make_async_remote_copy(
                src_ref=out_ref.at[rows(own_base + send_off, s)],
                dst_ref=out_ref.at[rows(other_base + recv_off, s)],
                send_sem=y_send_sems.at[s],
                recv_sem=y_recv_sems.at[s],
                device_id=y_peer,
                device_id_type=pl.DeviceIdType.MESH,
            ))

        for s in range(S):
            out_ref[rows(own_base + recv_off, s)] = x_ref[
                rows(recv_off, s)[0], :
            ].astype(jnp.bfloat16)

        for s in range(S):
            x_recv_d[s].wait_recv()
            d = pltpu.make_async_remote_copy(
                src_ref=out_ref.at[rows(other_base + fwd_off, s)],
                dst_ref=out_ref.at[rows(other_base + fwd_off, s)],
                send_sem=y_send_sems.at[s],
                recv_sem=y_recv_sems.at[s],
                device_id=y_peer,
                device_id_type=pl.DeviceIdType.MESH,
            )
            d.start()
            y_send_d.append(d)

        for s in range(S):
            y_recv_d[s].wait_recv()

        for s in range(S):
            x_send_d[s].wait_send()
            y_send_d[s].wait_send()

    return pl.pallas_call(
        body,
        out_shape=jax.ShapeDtypeStruct((2 * m, n), jnp.bfloat16),
        in_specs=[pl.BlockSpec(memory_space=pltpu.VMEM)],
        out_specs=pl.BlockSpec(memory_space=pltpu.VMEM),
        scratch_shapes=[
            pltpu.SemaphoreType.DMA((S,)),
            pltpu.SemaphoreType.DMA((S,)),
            pltpu.SemaphoreType.DMA((S,)),
            pltpu.SemaphoreType.DMA((S,)),
        ],
        compiler_params=pltpu.CompilerParams(collective_id=0),
    )(x)
